# Initial kernel scaffold; baseline (speedup 1.0000x reference)
import jax
import jax.numpy as jnp
from jax.experimental import pallas as pl


def kernel(h, edge_index):
    raise NotImplementedError("write your pallas kernel here")


# speedup vs baseline: 2.6070x; 2.6070x over previous
import functools

import jax
import jax.numpy as jnp
from jax import lax
from jax.experimental import pallas as pl
from jax.experimental.pallas import tpu as pltpu
from jax.experimental.pallas import tpu_sc as plsc

NC = 2
NS = 16
NW = NC * NS
LANES = 16


def _make_sc_kernel(n_nodes: int, d_feat: int, n_edges: int, chunk: int):
    assert n_edges % NW == 0
    e_per_w = n_edges // NW
    assert e_per_w % chunk == 0 and chunk % LANES == 0 and chunk % 8 == 0
    assert chunk <= 128
    n_steps = e_per_w // chunk
    n_groups = chunk // LANES
    n_k = d_feat // LANES

    mesh = plsc.VectorSubcoreMesh(
        core_axis_name="c", subcore_axis_name="s",
        num_cores=NC, num_subcores=NS)

    @functools.partial(
        pl.kernel,
        out_type=jax.ShapeDtypeStruct((n_edges,), jnp.float32),
        mesh=mesh,
        compiler_params=pltpu.CompilerParams(needs_layout_passes=False),
        scratch_types=[
            pltpu.VMEM((chunk,), jnp.int32),
            pltpu.VMEM((chunk,), jnp.int32),
            pltpu.VMEM((chunk, d_feat), jnp.float32),
            pltpu.VMEM((chunk, d_feat), jnp.float32),
            pltpu.VMEM((chunk,), jnp.float32),
            pltpu.SemaphoreType.DMA,
            pltpu.SemaphoreType.DMA,
        ],
    )
    def sc_kernel(h_hbm, src_hbm, dst_hbm, out_hbm,
                  idx_s, idx_d, rows_s, rows_d, scores, sem_s, sem_d):
        wid = lax.axis_index("s") * NC + lax.axis_index("c")
        lane = lax.broadcasted_iota(jnp.int32, (LANES,), 0)

        def step(i, carry):
            base = wid * e_per_w + i * chunk
            pltpu.sync_copy(src_hbm.at[pl.ds(base, chunk)], idx_s)
            pltpu.sync_copy(dst_hbm.at[pl.ds(base, chunk)], idx_d)
            cs = pltpu.async_copy(h_hbm.at[idx_s], rows_s, sem_s)
            cd = pltpu.async_copy(h_hbm.at[idx_d], rows_d, sem_d)
            cs.wait()
            cd.wait()

            def group(g, gcarry):
                vec = jnp.zeros((LANES,), jnp.float32)
                for j in range(LANES):
                    e = g * LANES + j
                    acc = rows_s[e, pl.ds(0, LANES)] * rows_d[e, pl.ds(0, LANES)]
                    for k in range(1, n_k):
                        acc = acc + (rows_s[e, pl.ds(k * LANES, LANES)]
                                     * rows_d[e, pl.ds(k * LANES, LANES)])
                    s = jnp.sum(acc)
                    vec = jnp.where(lane == j, s, vec)
                scores[pl.ds(g * LANES, LANES)] = vec
                return gcarry

            lax.fori_loop(0, n_groups, group, 0)
            pltpu.sync_copy(scores, out_hbm.at[pl.ds(base, chunk)])
            return carry

        lax.fori_loop(0, n_steps, step, 0)

    return sc_kernel


def kernel(h, edge_index):
    n_nodes, d_feat = h.shape
    n_edges = edge_index.shape[1]
    ei = edge_index.astype(jnp.int32)
    sc = _make_sc_kernel(n_nodes, d_feat, n_edges, chunk=80)
    return sc(h, ei[0], ei[1])
